# Initial kernel scaffold; baseline (speedup 1.0000x reference)
import jax
import jax.numpy as jnp
from jax.experimental import pallas as pl


def kernel(A, noise, idx):
    raise NotImplementedError("write your pallas kernel here")


# speedup vs baseline: 12.2070x; 12.2070x over previous
import functools

import jax
import jax.numpy as jnp
from jax.experimental import pallas as pl

_K = 32
_BLOCK_R = 128


def _topk_mask_body(a_ref, n_ref, o_ref, *, k):
    a = a_ref[...]
    adj = jnp.maximum(a, 0.0)
    s = adj + n_ref[...]
    v = jax.lax.bitcast_convert_type(s, jnp.int32)

    rows = v.shape[0]

    def bit_step(i, p):
        b = 30 - i
        cand = p | jnp.left_shift(jnp.int32(1), b)
        cnt = jnp.sum((v >= cand).astype(jnp.int32), axis=1, keepdims=True)
        return jnp.where(cnt >= k, cand, p)

    p0 = jnp.zeros((rows, 1), jnp.int32)
    p = jax.lax.fori_loop(0, 31, bit_step, p0)

    o_ref[...] = jnp.where(v >= p, adj, 0.0)


def kernel(A, noise, idx):
    del idx
    n_rows, n_cols = A.shape
    grid = (pl.cdiv(n_rows, _BLOCK_R),)
    out = pl.pallas_call(
        functools.partial(_topk_mask_body, k=_K),
        grid=grid,
        in_specs=[
            pl.BlockSpec((_BLOCK_R, n_cols), lambda i: (i, 0)),
            pl.BlockSpec((_BLOCK_R, n_cols), lambda i: (i, 0)),
        ],
        out_specs=pl.BlockSpec((_BLOCK_R, n_cols), lambda i: (i, 0)),
        out_shape=jax.ShapeDtypeStruct((n_rows, n_cols), A.dtype),
    )(A, noise)
    return out
